# Initial kernel scaffold; baseline (speedup 1.0000x reference)
import jax
import jax.numpy as jnp
from jax.experimental import pallas as pl


def kernel(x_t, x_s, edge_index_t, edge_weight_t, edge_index_s, edge_weight_s, edge_index_t1, edge_weight_t1, edge_index_s1, edge_weight_s1, ntc0_k1, ntc0_k3, ntc0_k5, ntc1_red, ntc1_k1, ntc1_k3, ntc1_k5, nsc0_W, nsc1_W, nbn0_g, nbn0_b, nbn1_g, nbn1_b, esc0_W, esc0_bng, esc0_bnb, esc1_W, esc1_bng, esc1_bnb, esc2_W, esc2_bng, esc2_bnb, lin1_W, lin1_b, bn1_g, bn1_b, lin2_W, lin2_b, bn2_g, bn2_b, lin3_W, lin3_b):
    raise NotImplementedError("write your pallas kernel here")


# speedup vs baseline: 1.1575x; 1.1575x over previous
import jax
import jax.numpy as jnp
from jax.experimental import pallas as pl

_B = 8
_T = 375


def _mm(A, B, extras=(), alpha=1.0, relu=False, bm=512):
    M, K = A.shape
    N = B.shape[1]
    if M <= bm:
        bm = M
    Mp = ((M + bm - 1) // bm) * bm
    coefs = tuple(float(c) for c, _ in extras)
    arrs = [jnp.pad(E, ((0, Mp - M), (0, 0))) if Mp != M else E for _, E in extras]
    Ap = jnp.pad(A, ((0, Mp - M), (0, 0))) if Mp != M else A

    def kern(a_ref, b_ref, *refs):
        o_ref = refs[-1]
        acc = jnp.dot(a_ref[...], b_ref[...], preferred_element_type=jnp.float32, precision=jax.lax.Precision.HIGHEST)
        if alpha != 1.0:
            acc = alpha * acc
        for c, r in zip(coefs, refs[:-1]):
            acc = acc + c * r[...]
        if relu:
            acc = jnp.maximum(acc, 0.0)
        o_ref[...] = acc

    in_specs = [pl.BlockSpec((bm, K), lambda i: (i, 0)),
                pl.BlockSpec((K, N), lambda i: (0, 0))]
    in_specs += [pl.BlockSpec((bm, N), lambda i: (i, 0)) for _ in arrs]
    out = pl.pallas_call(
        kern, grid=(Mp // bm,), in_specs=in_specs,
        out_specs=pl.BlockSpec((bm, N), lambda i: (i, 0)),
        out_shape=jax.ShapeDtypeStruct((Mp, N), jnp.float32),
    )(Ap, B, *arrs)
    return out[:M] if Mp != M else out


def _bn_act(x, g, b, slope):
    N, F = x.shape

    def kern(x_ref, g_ref, b_ref, o_ref):
        xv = x_ref[...]
        mu = jnp.mean(xv, axis=0, keepdims=True)
        xc = xv - mu
        var = jnp.mean(xc * xc, axis=0, keepdims=True)
        y = xc * jax.lax.rsqrt(var + 1e-5) * g_ref[...] + b_ref[...]
        o_ref[...] = jnp.where(y >= 0.0, y, slope * y)

    return pl.pallas_call(
        kern, out_shape=jax.ShapeDtypeStruct((N, F), jnp.float32),
    )(x, g.reshape(1, F), b.reshape(1, F))


def _pairmax(x, bm=4096):
    N, _, F = x.shape
    if N <= bm:
        bm = N
    Np = ((N + bm - 1) // bm) * bm
    xp = jnp.pad(x, ((0, Np - N), (0, 0), (0, 0))) if Np != N else x

    def kern(x_ref, o_ref):
        o_ref[...] = jnp.maximum(x_ref[:, 0, :], x_ref[:, 1, :])

    out = pl.pallas_call(
        kern, grid=(Np // bm,),
        in_specs=[pl.BlockSpec((bm, 2, F), lambda i: (i, 0, 0))],
        out_specs=pl.BlockSpec((bm, F), lambda i: (i, 0)),
        out_shape=jax.ShapeDtypeStruct((Np, F), jnp.float32))(xp)
    return out[:N] if Np != N else out


def _lincomb(terms, bm=4096):
    coefs = [float(c) for c, _ in terms]
    arrs = [a for _, a in terms]
    N, F = arrs[0].shape
    if N <= bm:
        bm = N
    Np = ((N + bm - 1) // bm) * bm
    if Np != N:
        arrs = [jnp.pad(a, ((0, Np - N), (0, 0))) for a in arrs]

    def kern(*refs):
        o_ref = refs[-1]
        acc = coefs[0] * refs[0][...]
        for c, r in zip(coefs[1:], refs[1:-1]):
            acc = acc + c * r[...]
        o_ref[...] = acc

    out = pl.pallas_call(
        kern, grid=(Np // bm,),
        in_specs=[pl.BlockSpec((bm, F), lambda i: (i, 0)) for _ in arrs],
        out_specs=pl.BlockSpec((bm, F), lambda i: (i, 0)),
        out_shape=jax.ShapeDtypeStruct((Np, F), jnp.float32),
    )(*arrs)
    return out[:N] if Np != N else out


def _bn_act_T(x, g, b, slope):
    N, F = x.shape

    def kern(x_ref, g_ref, b_ref, o_ref):
        xv = x_ref[...]
        mu = jnp.mean(xv, axis=1, keepdims=True)
        xc = xv - mu
        var = jnp.mean(xc * xc, axis=1, keepdims=True)
        y = xc * jax.lax.rsqrt(var + 1e-5) * g_ref[...] + b_ref[...]
        o_ref[...] = jnp.where(y >= 0.0, y, slope * y)

    out = pl.pallas_call(
        kern, out_shape=jax.ShapeDtypeStruct((F, N), jnp.float32),
    )(x.T, g.reshape(F, 1), b.reshape(F, 1))
    return out.T


def _mean_axis1(x):
    N, Tm, F = x.shape

    def kern(x_ref, o_ref):
        o_ref[...] = jnp.mean(x_ref[...], axis=1)

    return pl.pallas_call(
        kern, out_shape=jax.ShapeDtypeStruct((N, F), jnp.float32))(x)


def _mlp(x, W1, b1, g1, bb1, W2, b2, g2, bb2, W3, b3):
    Bm = x.shape[0]

    def bn(h, g, b):
        mu = jnp.mean(h, axis=0, keepdims=True)
        hc = h - mu
        var = jnp.mean(hc * hc, axis=0, keepdims=True)
        return hc * jax.lax.rsqrt(var + 1e-5) * g + b

    def kern(x_ref, w1, b1r, g1r, bb1r, w2, b2r, g2r, bb2r, w3, b3r, o_ref):
        h = jnp.dot(x_ref[...], w1[...], preferred_element_type=jnp.float32, precision=jax.lax.Precision.HIGHEST) + b1r[...]
        h = jnp.maximum(bn(h, g1r[...], bb1r[...]), 0.0)
        h = jnp.dot(h, w2[...], preferred_element_type=jnp.float32, precision=jax.lax.Precision.HIGHEST) + b2r[...]
        h = jnp.maximum(bn(h, g2r[...], bb2r[...]), 0.0)
        o_ref[...] = jnp.dot(h, w3[...], preferred_element_type=jnp.float32, precision=jax.lax.Precision.HIGHEST) + b3r[...]

    return pl.pallas_call(
        kern, out_shape=jax.ShapeDtypeStruct((Bm, W3.shape[1]), jnp.float32),
    )(x, W1, b1.reshape(1, -1), g1.reshape(1, -1), bb1.reshape(1, -1),
      W2, b2.reshape(1, -1), g2.reshape(1, -1), bb2.reshape(1, -1),
      W3, b3.reshape(1, -1))


def _prop_sparse(x, src, dst, ew):
    m = x[src] * ew[:, None]
    return jnp.zeros_like(x).at[dst].add(m)


def _cheb_dense(Xf, A, W, fin, fout):
    Nn = Xf.shape[0]
    out = _mm(Xf.reshape(-1, fin), W[0])
    Tx1 = _mm(A, Xf)
    out = _mm(Tx1.reshape(-1, fin), W[1], extras=[(1.0, out)])
    Tx0 = Xf
    for k in range(2, W.shape[0]):
        Tx2 = _mm(A, Tx1, alpha=2.0, extras=[(-1.0, Tx0)])
        out = _mm(Tx2.reshape(-1, fin), W[k], extras=[(1.0, out)])
        Tx0, Tx1 = Tx1, Tx2
    return out.reshape(Nn, -1, fout)


def _laguerre(x, src, dst, ew, W):
    out = _mm(x, W[0])
    Px = _prop_sparse(x, src, dst, ew)
    Tx1 = _lincomb([(1.0, x), (-1.0, Px)])
    out = _mm(Tx1, W[1], extras=[(1.0, out)])
    Tx0 = x
    for k in range(2, W.shape[0]):
        j = k - 1.0
        P1 = _prop_sparse(Tx1, src, dst, ew)
        Tx2 = _lincomb([((2.0 * j + 1.0) / (j + 1.0), Tx1),
                        (-1.0 / (j + 1.0), P1),
                        (-j / (j + 1.0), Tx0)])
        out = _mm(Tx2, W[k], extras=[(1.0, out)])
        Tx0, Tx1 = Tx1, Tx2
    return out


def _im2col_stride4(x, k_lo, k_hi, n_out):
    xp = jnp.pad(x, ((0, 0), (k_lo, k_hi)) + ((0, 0),) * (x.ndim - 2))
    taps = k_lo + k_hi + 1
    cols = [xp[:, d:d + 4 * (n_out - 1) + 1:4] for d in range(taps)]
    return jnp.stack(cols, axis=2)


def kernel(x_t, x_s, edge_index_t, edge_weight_t, edge_index_s, edge_weight_s,
           edge_index_t1, edge_weight_t1, edge_index_s1, edge_weight_s1,
           ntc0_k1, ntc0_k3, ntc0_k5, ntc1_red, ntc1_k1, ntc1_k3, ntc1_k5,
           nsc0_W, nsc1_W, nbn0_g, nbn0_b, nbn1_g, nbn1_b,
           esc0_W, esc0_bng, esc0_bnb, esc1_W, esc1_bng, esc1_bnb,
           esc2_W, esc2_bng, esc2_bnb,
           lin1_W, lin1_b, bn1_g, bn1_b, lin2_W, lin2_b, bn2_g, bn2_b,
           lin3_W, lin3_b):
    NT = x_t.shape[0]
    NS = x_s.shape[0]
    NT1 = NT // 2

    A_t = jnp.zeros((NT, NT), jnp.float32).at[
        edge_index_t[1], edge_index_t[0]].add(edge_weight_t)
    A_t1 = jnp.zeros((NT1, NT1), jnp.float32).at[
        edge_index_t1[1], edge_index_t1[0]].add(edge_weight_t1)

    t1 = (_T + 3) // 4
    cols0 = _im2col_stride4(x_t, 1, 3, t1)
    W0 = jnp.zeros((5, 12), jnp.float32)
    W0 = W0.at[1, 0:4].set(ntc0_k1[0, 0])
    W0 = W0.at[1:4, 4:8].set(ntc0_k3[:, 0, :])
    W0 = W0.at[0:5, 8:12].set(ntc0_k5[:, 0, :])
    h = _mm(cols0.reshape(-1, 5), W0, relu=True)
    h = h.reshape(NT, t1 * 12)

    h = _cheb_dense(h, A_t, nsc0_W, 12, 8)
    h = _pairmax(h.reshape(NT1, 2, t1 * 8))
    h = _bn_act(h, nbn0_g, nbn0_b, 0.33)

    r = _mm(h.reshape(-1, 8), ntc1_red[0], relu=True)
    t2 = (t1 + 3) // 4
    cols1 = _im2col_stride4(r.reshape(NT1, t1, 4), 1, 3, t2)
    W1 = jnp.zeros((20, 24), jnp.float32)
    W1 = W1.at[4:8, 0:8].set(ntc1_k1[0])
    for d in range(3):
        W1 = W1.at[(d + 1) * 4:(d + 2) * 4, 8:16].set(ntc1_k3[d])
    for d in range(5):
        W1 = W1.at[d * 4:(d + 1) * 4, 16:24].set(ntc1_k5[d])
    h = _mm(cols1.reshape(-1, 20), W1, relu=True)
    h = h.reshape(NT1, t2 * 24)

    h = _cheb_dense(h, A_t1, nsc1_W, 24, 16)
    h = _bn_act(h.reshape(NT1, t2 * 16), nbn1_g, nbn1_b, 0.33)
    h = _mean_axis1(h.reshape(NT1, t2, 16))

    s = _laguerre(x_s, edge_index_s[0], edge_index_s[1], edge_weight_s, esc0_W)
    s = _bn_act_T(s, esc0_bng, esc0_bnb, 0.33)
    s = _pairmax(s.reshape(NS // 2, 2, s.shape[1]))
    s = _laguerre(s, edge_index_s1[0], edge_index_s1[1], edge_weight_s1, esc1_W)
    s = _bn_act_T(s, esc1_bng, esc1_bnb, 0.33)
    s = _laguerre(s, edge_index_s1[0], edge_index_s1[1], edge_weight_s1, esc2_W)
    s = _bn_act_T(s, esc2_bng, esc2_bnb, 0.33)

    x = jnp.concatenate([h.reshape(_B, -1), s.reshape(_B, -1)], axis=-1)
    return _mlp(x, lin1_W, lin1_b, bn1_g, bn1_b,
                lin2_W, lin2_b, bn2_g, bn2_b, lin3_W, lin3_b)
